# Initial kernel scaffold; baseline (speedup 1.0000x reference)
import jax
import jax.numpy as jnp
from jax import lax
from jax.experimental import pallas as pl
from jax.experimental.pallas import tpu as pltpu
from jax.experimental.pallas import tpu_sc as plsc

_NUM_CLASSES = 128
_N_SUPPORT = 160000
_N_QUERY = 4096
_D_IN = 128
_D_EMB = 64

_CHUNK = 128
_NUM_CHUNKS = _N_SUPPORT // _CHUNK
_NW = 32
_K_MAX = -(-_NUM_CHUNKS // _NW)


def _sc_body(x_hbm, lbl_hbm, sums_hbm, cnt_hbm,
             buf, lblv, onesv, zer, zer16, sums_acc, cnt_acc):
    cid = lax.axis_index("c")
    sid = lax.axis_index("s")
    wid = sid * 2 + cid

    zero16 = jnp.zeros((16,), jnp.float32)
    one16 = jnp.ones((16,), jnp.float32)

    def _fill_zer(i, c):
        for j in range(_D_IN // 16):
            zer[i, pl.ds(j * 16, 16)] = zero16
        zer16[i, :] = zero16
        return c

    lax.fori_loop(0, 8, _fill_zer, 0)

    def _fill_ones(i, c):
        onesv[i, :] = one16
        return c

    lax.fori_loop(0, _CHUNK, _fill_ones, 0)

    pltpu.sync_copy(zer, sums_acc.at[pl.ds(sid * 8, 8)])
    pltpu.sync_copy(zer16, cnt_acc.at[pl.ds(sid * 8, 8)])
    plsc.subcore_barrier()

    def _chunk_step(k, c):
        chunk = k * _NW + wid

        @pl.when(chunk < _NUM_CHUNKS)
        def _():
            base = chunk * _CHUNK
            pltpu.sync_copy(x_hbm.at[pl.ds(base, _CHUNK)], buf)
            pltpu.sync_copy(lbl_hbm.at[pl.ds(base, _CHUNK)], lblv)
            pltpu.sync_copy(buf, sums_acc.at[lblv], add=True)
            pltpu.sync_copy(onesv, cnt_acc.at[lblv], add=True)

        return c

    lax.fori_loop(0, _K_MAX, _chunk_step, 0)
    plsc.subcore_barrier()

    @pl.when(sid == 0)
    def _():
        pltpu.sync_copy(sums_acc, sums_hbm.at[cid])
        pltpu.sync_copy(cnt_acc, cnt_hbm.at[cid])


def _make_sc_call():
    mesh = plsc.VectorSubcoreMesh(core_axis_name="c", subcore_axis_name="s")
    return pl.kernel(
        _sc_body,
        out_type=(
            jax.ShapeDtypeStruct((2, _NUM_CLASSES, _D_IN), jnp.float32),
            jax.ShapeDtypeStruct((2, _NUM_CLASSES, 16), jnp.float32),
        ),
        mesh=mesh,
        scratch_types=[
            pltpu.VMEM((_CHUNK, _D_IN), jnp.float32),
            pltpu.VMEM((_CHUNK,), jnp.int32),
            pltpu.VMEM((_CHUNK, 16), jnp.float32),
            pltpu.VMEM((8, _D_IN), jnp.float32),
            pltpu.VMEM((8, 16), jnp.float32),
            pltpu.VMEM_SHARED((_NUM_CLASSES, _D_IN), jnp.float32),
            pltpu.VMEM_SHARED((_NUM_CLASSES, 16), jnp.float32),
        ],
    )


def _tc_body(sums_ref, cnt_ref, w_ref, b_ref, q_ref, prob_ref, dist_ref):
    S = sums_ref[0] + sums_ref[1]
    cnt = cnt_ref[0] + cnt_ref[1]
    cntcol = cnt[:, 0:1]
    W = w_ref[...]
    b = b_ref[...]
    SW = jnp.dot(S, W, preferred_element_type=jnp.float32)
    proto = (SW + cntcol * b) / jnp.maximum(cntcol, 1.0)

    qe = jnp.dot(q_ref[...], W, preferred_element_type=jnp.float32) + b
    qn = jnp.sum(qe * qe, axis=1, keepdims=True)
    pn = jnp.sum(proto * proto, axis=1)[None, :]
    cross = jnp.dot(qe, proto.T, preferred_element_type=jnp.float32)
    d = qn + pn - 2.0 * cross
    dist_ref[...] = d
    nd = -d
    m = jnp.max(nd, axis=1, keepdims=True)
    e = jnp.exp(nd - m)
    prob_ref[...] = e / jnp.sum(e, axis=1, keepdims=True)


_BQ = 512


def _tc_call(sums, cnt, W, b2, query_set, interpret=False):
    grid = (_N_QUERY // _BQ,)
    return pl.pallas_call(
        _tc_body,
        grid=grid,
        in_specs=[
            pl.BlockSpec((2, _NUM_CLASSES, _D_IN), lambda i: (0, 0, 0)),
            pl.BlockSpec((2, _NUM_CLASSES, 16), lambda i: (0, 0, 0)),
            pl.BlockSpec((_D_IN, _D_EMB), lambda i: (0, 0)),
            pl.BlockSpec((1, _D_EMB), lambda i: (0, 0)),
            pl.BlockSpec((_BQ, _D_IN), lambda i: (i, 0)),
        ],
        out_specs=[
            pl.BlockSpec((_BQ, _NUM_CLASSES), lambda i: (i, 0)),
            pl.BlockSpec((_BQ, _NUM_CLASSES), lambda i: (i, 0)),
        ],
        out_shape=[
            jax.ShapeDtypeStruct((_N_QUERY, _NUM_CLASSES), jnp.float32),
            jax.ShapeDtypeStruct((_N_QUERY, _NUM_CLASSES), jnp.float32),
        ],
        interpret=interpret,
    )(sums, cnt, W, b2, query_set)


def kernel(support_set, support_labels, query_set, W, b):
    labels = support_labels.astype(jnp.int32)
    sums, cnt = _make_sc_call()(support_set, labels)
    prob, dist = _tc_call(sums, cnt, W, b.reshape(1, _D_EMB), query_set)
    class_labels = jnp.arange(_NUM_CLASSES, dtype=support_labels.dtype)
    return (prob, class_labels, dist)


# speedup vs baseline: 5.7594x; 5.7594x over previous
import jax
import jax.numpy as jnp
from jax import lax
from jax.experimental import pallas as pl
from jax.experimental.pallas import tpu as pltpu
from jax.experimental.pallas import tpu_sc as plsc

_NUM_CLASSES = 128
_N_SUPPORT = 160000
_N_QUERY = 4096
_D_IN = 128
_D_EMB = 64

_CHUNK = 128
_NUM_CHUNKS = _N_SUPPORT // _CHUNK
_NW = 32
_K_MAX = -(-_NUM_CHUNKS // _NW)


def _sc_body(x_hbm, lbl_hbm, sums_hbm, cnt_hbm,
             buf, lblv, onesv, zer, sums_acc, cnt_acc):
    cid = lax.axis_index("c")
    sid = lax.axis_index("s")
    wid = sid * 2 + cid

    zero16 = jnp.zeros((16,), jnp.float32)
    one16 = jnp.ones((16,), jnp.float32)

    def _fill_zer(i, c):
        for j in range(_D_IN // 16):
            zer[i, pl.ds(j * 16, 16)] = zero16
        return c

    lax.fori_loop(0, 8, _fill_zer, 0)

    def _fill_ones(i, c):
        for j in range(_D_IN // 16):
            onesv[i, pl.ds(j * 16, 16)] = one16
        return c

    lax.fori_loop(0, _CHUNK, _fill_ones, 0)

    pltpu.sync_copy(zer, sums_acc.at[pl.ds(sid * 8, 8)])
    pltpu.sync_copy(zer, cnt_acc.at[pl.ds(sid * 8, 8)])
    plsc.subcore_barrier()

    def _chunk_step(k, c):
        chunk = k * _NW + wid

        @pl.when(chunk < _NUM_CHUNKS)
        def _():
            base = chunk * _CHUNK
            pltpu.sync_copy(x_hbm.at[pl.ds(base, _CHUNK)], buf)
            pltpu.sync_copy(lbl_hbm.at[pl.ds(base, _CHUNK)], lblv)
            pltpu.sync_copy(buf, sums_acc.at[lblv], add=True)
            pltpu.sync_copy(onesv, cnt_acc.at[lblv], add=True)

        return c

    lax.fori_loop(0, _K_MAX, _chunk_step, 0)
    plsc.subcore_barrier()

    @pl.when(sid == 0)
    def _():
        pltpu.sync_copy(sums_acc, sums_hbm.at[cid])
        pltpu.sync_copy(cnt_acc, cnt_hbm.at[cid])


def _make_sc_call():
    mesh = plsc.VectorSubcoreMesh(core_axis_name="c", subcore_axis_name="s")
    return pl.kernel(
        _sc_body,
        out_type=(
            jax.ShapeDtypeStruct((2, _NUM_CLASSES, _D_IN), jnp.float32),
            jax.ShapeDtypeStruct((2, _NUM_CLASSES, _D_IN), jnp.float32),
        ),
        mesh=mesh,
        scratch_types=[
            pltpu.VMEM((_CHUNK, _D_IN), jnp.float32),
            pltpu.VMEM((_CHUNK,), jnp.int32),
            pltpu.VMEM((_CHUNK, _D_IN), jnp.float32),
            pltpu.VMEM((8, _D_IN), jnp.float32),
            pltpu.VMEM_SHARED((_NUM_CLASSES, _D_IN), jnp.float32),
            pltpu.VMEM_SHARED((_NUM_CLASSES, _D_IN), jnp.float32),
        ],
    )


def _tc_body(sums_ref, cnt_ref, w_ref, b_ref, q_ref, prob_ref, dist_ref):
    S = sums_ref[0] + sums_ref[1]
    cnt = cnt_ref[0] + cnt_ref[1]
    cntcol = cnt[:, 0:1]
    W = w_ref[...]
    b = b_ref[...]
    SW = jnp.dot(S, W, preferred_element_type=jnp.float32)
    proto = (SW + cntcol * b) / jnp.maximum(cntcol, 1.0)

    qe = jnp.dot(q_ref[...], W, preferred_element_type=jnp.float32) + b
    qn = jnp.sum(qe * qe, axis=1, keepdims=True)
    pn = jnp.sum(proto * proto, axis=1)[None, :]
    cross = jnp.dot(qe, proto.T, preferred_element_type=jnp.float32)
    d = qn + pn - 2.0 * cross
    dist_ref[...] = d
    nd = -d
    m = jnp.max(nd, axis=1, keepdims=True)
    e = jnp.exp(nd - m)
    prob_ref[...] = e / jnp.sum(e, axis=1, keepdims=True)


_BQ = 512


def _tc_call(sums, cnt, W, b2, query_set, interpret=False):
    grid = (_N_QUERY // _BQ,)
    return pl.pallas_call(
        _tc_body,
        grid=grid,
        in_specs=[
            pl.BlockSpec((2, _NUM_CLASSES, _D_IN), lambda i: (0, 0, 0)),
            pl.BlockSpec((2, _NUM_CLASSES, _D_IN), lambda i: (0, 0, 0)),
            pl.BlockSpec((_D_IN, _D_EMB), lambda i: (0, 0)),
            pl.BlockSpec((1, _D_EMB), lambda i: (0, 0)),
            pl.BlockSpec((_BQ, _D_IN), lambda i: (i, 0)),
        ],
        out_specs=[
            pl.BlockSpec((_BQ, _NUM_CLASSES), lambda i: (i, 0)),
            pl.BlockSpec((_BQ, _NUM_CLASSES), lambda i: (i, 0)),
        ],
        out_shape=[
            jax.ShapeDtypeStruct((_N_QUERY, _NUM_CLASSES), jnp.float32),
            jax.ShapeDtypeStruct((_N_QUERY, _NUM_CLASSES), jnp.float32),
        ],
        interpret=interpret,
    )(sums, cnt, W, b2, query_set)


def kernel(support_set, support_labels, query_set, W, b):
    labels = support_labels.astype(jnp.int32)
    sums, cnt = _make_sc_call()(support_set, labels)
    prob, dist = _tc_call(sums, cnt, W, b.reshape(1, _D_EMB), query_set)
    class_labels = jnp.arange(_NUM_CLASSES, dtype=support_labels.dtype)
    return (prob, class_labels, dist)


# speedup vs baseline: 11.8816x; 2.0630x over previous
import jax
import jax.numpy as jnp
from jax import lax
from jax.experimental import pallas as pl
from jax.experimental.pallas import tpu as pltpu
from jax.experimental.pallas import tpu_sc as plsc

_NUM_CLASSES = 128
_N_SUPPORT = 160000
_N_QUERY = 4096
_D_IN = 128
_D_EMB = 64

_BLK = 256
_NBLK = _N_SUPPORT // _BLK
_NW = 32
_K_MAX = -(-_NBLK // _NW)


def _sc_body(x_hbm, lbl_hbm, sums_hbm,
             buf0, buf1, la0, lb0, la1, lb1, zer, sums_acc, sem0, sem1):
    cid = lax.axis_index("c")
    sid = lax.axis_index("s")
    wid = sid * 2 + cid

    zero16 = jnp.zeros((16,), jnp.float32)

    def _fill_zer(i, c):
        for j in range(_D_IN // 16):
            zer[i, pl.ds(j * 16, 16)] = zero16
        return c

    lax.fori_loop(0, 8, _fill_zer, 0)

    pltpu.sync_copy(zer, sums_acc.at[pl.ds(sid * 8, 8)])
    plsc.subcore_barrier()

    bufs = (buf0, buf1)
    lbls = ((la0, lb0), (la1, lb1))
    sems = (sem0, sem1)

    def _issue(k, slot):
        blk = k * _NW + wid

        @pl.when(blk < _NBLK)
        def _():
            base = blk * _BLK
            pltpu.async_copy(x_hbm.at[pl.ds(base, _BLK)], bufs[slot], sems[slot])
            pltpu.async_copy(lbl_hbm.at[pl.ds(base, 128)], lbls[slot][0], sems[slot])
            pltpu.async_copy(lbl_hbm.at[pl.ds(base + 128, 128)], lbls[slot][1], sems[slot])

    def _drain_scatter(k, slot):
        blk = k * _NW + wid

        @pl.when(blk < _NBLK)
        def _():
            base = blk * _BLK
            pltpu.make_async_copy(
                x_hbm.at[pl.ds(base, _BLK)], bufs[slot], sems[slot]).wait()
            pltpu.make_async_copy(
                lbl_hbm.at[pl.ds(base, 128)], lbls[slot][0], sems[slot]).wait()
            pltpu.make_async_copy(
                lbl_hbm.at[pl.ds(base + 128, 128)], lbls[slot][1], sems[slot]).wait()
            pltpu.sync_copy(bufs[slot].at[pl.ds(0, 128)],
                            sums_acc.at[lbls[slot][0]], add=True)
            pltpu.sync_copy(bufs[slot].at[pl.ds(128, 128)],
                            sums_acc.at[lbls[slot][1]], add=True)

    for slot in range(2):
        _issue(slot, slot)

    def _step(kk, c):
        for slot in range(2):
            k = kk * 2 + slot
            _drain_scatter(k, slot)
            _issue(k + 2, slot)
        return c

    lax.fori_loop(0, _K_MAX // 2, _step, 0)
    plsc.subcore_barrier()

    @pl.when(sid == 0)
    def _():
        pltpu.sync_copy(sums_acc, sums_hbm.at[cid])


def _make_sc_call():
    mesh = plsc.VectorSubcoreMesh(core_axis_name="c", subcore_axis_name="s")
    return pl.kernel(
        _sc_body,
        out_type=jax.ShapeDtypeStruct((2, _NUM_CLASSES, _D_IN), jnp.float32),
        mesh=mesh,
        scratch_types=[
            pltpu.VMEM((_BLK, _D_IN), jnp.float32),
            pltpu.VMEM((_BLK, _D_IN), jnp.float32),
            pltpu.VMEM((128,), jnp.int32),
            pltpu.VMEM((128,), jnp.int32),
            pltpu.VMEM((128,), jnp.int32),
            pltpu.VMEM((128,), jnp.int32),
            pltpu.VMEM((8, _D_IN), jnp.float32),
            pltpu.VMEM_SHARED((_NUM_CLASSES, _D_IN), jnp.float32),
            pltpu.SemaphoreType.DMA,
            pltpu.SemaphoreType.DMA,
        ],
    )


_CB = 2048
_CROWS = 79


def _cnt_body(lab_ref, cnt_ref):
    i = pl.program_id(0)
    lab = lab_ref[0]
    iota = lax.broadcasted_iota(jnp.int32, (_NUM_CLASSES, _CB), 0)
    oh = jnp.where(lab == iota, 1.0, 0.0)
    part = jnp.dot(oh, jnp.ones((_CB, 8), jnp.float32),
                   preferred_element_type=jnp.float32)

    @pl.when(i == 0)
    def _():
        cnt_ref[...] = part

    @pl.when(i > 0)
    def _():
        cnt_ref[...] += part


def _cnt_call(labf, interpret=False):
    return pl.pallas_call(
        _cnt_body,
        grid=(_CROWS,),
        in_specs=[pl.BlockSpec((1, 1, _CB), lambda i: (i, 0, 0))],
        out_specs=pl.BlockSpec((_NUM_CLASSES, 8), lambda i: (0, 0)),
        out_shape=jax.ShapeDtypeStruct((_NUM_CLASSES, 8), jnp.float32),
        interpret=interpret,
    )(labf)


def _tc_body(sums_ref, cnt_ref, w_ref, b_ref, q_ref, prob_ref, dist_ref):
    S = sums_ref[0] + sums_ref[1]
    cntcol = cnt_ref[:, 0:1]
    W = w_ref[...]
    b = b_ref[...]
    SW = jnp.dot(S, W, preferred_element_type=jnp.float32)
    proto = (SW + cntcol * b) / jnp.maximum(cntcol, 1.0)

    qe = jnp.dot(q_ref[...], W, preferred_element_type=jnp.float32) + b
    qn = jnp.sum(qe * qe, axis=1, keepdims=True)
    pn = jnp.sum(proto * proto, axis=1)[None, :]
    cross = jnp.dot(qe, proto.T, preferred_element_type=jnp.float32)
    d = qn + pn - 2.0 * cross
    dist_ref[...] = d
    nd = -d
    m = jnp.max(nd, axis=1, keepdims=True)
    e = jnp.exp(nd - m)
    prob_ref[...] = e / jnp.sum(e, axis=1, keepdims=True)


_BQ = 512


def _tc_call(sums, cnt, W, b2, query_set, interpret=False):
    grid = (_N_QUERY // _BQ,)
    return pl.pallas_call(
        _tc_body,
        grid=grid,
        in_specs=[
            pl.BlockSpec((2, _NUM_CLASSES, _D_IN), lambda i: (0, 0, 0)),
            pl.BlockSpec((_NUM_CLASSES, 8), lambda i: (0, 0)),
            pl.BlockSpec((_D_IN, _D_EMB), lambda i: (0, 0)),
            pl.BlockSpec((1, _D_EMB), lambda i: (0, 0)),
            pl.BlockSpec((_BQ, _D_IN), lambda i: (i, 0)),
        ],
        out_specs=[
            pl.BlockSpec((_BQ, _NUM_CLASSES), lambda i: (i, 0)),
            pl.BlockSpec((_BQ, _NUM_CLASSES), lambda i: (i, 0)),
        ],
        out_shape=[
            jax.ShapeDtypeStruct((_N_QUERY, _NUM_CLASSES), jnp.float32),
            jax.ShapeDtypeStruct((_N_QUERY, _NUM_CLASSES), jnp.float32),
        ],
        interpret=interpret,
    )(sums, cnt, W, b2, query_set)


def kernel(support_set, support_labels, query_set, W, b):
    labels = support_labels.astype(jnp.int32)
    labf = jnp.pad(labels, (0, _CROWS * _CB - _N_SUPPORT),
                   constant_values=1 << 30).reshape(_CROWS, 1, _CB)
    cnt = _cnt_call(labf)
    sums = _make_sc_call()(support_set, labels)
    prob, dist = _tc_call(sums, cnt, W, b.reshape(1, _D_EMB), query_set)
    class_labels = jnp.arange(_NUM_CLASSES, dtype=support_labels.dtype)
    return (prob, class_labels, dist)
